# baseline (speedup 1.0000x reference)
import functools

import jax
import jax.numpy as jnp
from jax import lax
from jax.experimental import pallas as pl
from jax.experimental.pallas import tpu as pltpu

K = 2
SUB = 128
BF = 256


def _mlp_body(pstart_ref, counts_ref, x_ref, w1_ref, w3_ref, w2_ref, o_ref):
    e = pl.program_id(0)
    f = pl.program_id(1)
    p0 = pstart_ref[e]
    cnt = counts_ref[e]
    nsub = (cnt + SUB - 1) // SUB
    w1 = w1_ref[0]
    w3 = w3_ref[0]
    w2 = w2_ref[0]

    def body(j, carry):
        off = pl.multiple_of(p0 + j * SUB, SUB)
        x = x_ref[pl.ds(off, SUB), :]
        gate = lax.dot_general(x, w1, (((1,), (1,)), ((), ())),
                               preferred_element_type=jnp.float32,
                               precision=lax.Precision.DEFAULT)
        up = lax.dot_general(x, w3, (((1,), (1,)), ((), ())),
                             preferred_element_type=jnp.float32,
                             precision=lax.Precision.DEFAULT)
        h = gate * jax.nn.sigmoid(gate) * up
        part = lax.dot_general(h, w2, (((1,), (1,)), ((), ())),
                               preferred_element_type=jnp.float32,
                               precision=lax.Precision.DEFAULT)

        @pl.when(f == 0)
        def _():
            o_ref[pl.ds(off, SUB), :] = part

        @pl.when(f > 0)
        def _():
            o_ref[pl.ds(off, SUB), :] += part

        return carry

    lax.fori_loop(0, nsub, body, 0)


def _grouped_mlp(x_sorted, w13, w2, pstart, counts, n_experts, d_ff):
    m_pad, d_model = x_sorted.shape
    nf = d_ff // BF
    grid_spec = pltpu.PrefetchScalarGridSpec(
        num_scalar_prefetch=2,
        grid=(n_experts, nf),
        in_specs=[
            pl.BlockSpec((m_pad, d_model), lambda e, f, ps, ct: (0, 0)),
            pl.BlockSpec((1, BF, d_model), lambda e, f, ps, ct: (e, f, 0)),
            pl.BlockSpec((1, BF, d_model), lambda e, f, ps, ct: (e, nf + f, 0)),
            pl.BlockSpec((1, d_model, BF), lambda e, f, ps, ct: (e, 0, f)),
        ],
        out_specs=pl.BlockSpec((m_pad, d_model), lambda e, f, ps, ct: (0, 0)),
    )
    return pl.pallas_call(
        _mlp_body,
        grid_spec=grid_spec,
        out_shape=jax.ShapeDtypeStruct((m_pad, d_model), jnp.float32),
        compiler_params=pltpu.CompilerParams(
            dimension_semantics=("arbitrary", "arbitrary"),
        ),
    )(pstart, counts, x_sorted, w13, w13, w2)


def kernel(hidden_states, use_grouped_topk, top_k, router_logits,
           renormalize, W13, W2):
    t, d_model = hidden_states.shape
    e = router_logits.shape[1]
    d_ff = W2.shape[2]
    m = t * K
    m_pad = m + e * SUB

    probs = jax.nn.softmax(router_logits.astype(jnp.float32), axis=-1)
    topk_w, topk_idx = lax.top_k(probs, K)
    denom = jnp.sum(topk_w, axis=-1, keepdims=True)
    topk_w = jnp.where(jnp.asarray(renormalize), topk_w / denom, topk_w)
    topk_w = topk_w * (jnp.asarray(1, jnp.float32)
                       - jnp.asarray(use_grouped_topk, jnp.float32))

    e_flat = topk_idx.reshape(-1).astype(jnp.int32)
    sort_idx = jnp.argsort(e_flat, stable=True).astype(jnp.int32)
    tok_sorted = (sort_idx // K).astype(jnp.int32)
    e_sorted = e_flat[sort_idx]
    counts = jnp.bincount(e_flat, length=e).astype(jnp.int32)
    padded = ((counts + SUB - 1) // SUB) * SUB
    pstart = jnp.concatenate([jnp.zeros((1,), padded.dtype),
                              jnp.cumsum(padded)[:-1]])
    start = jnp.concatenate([jnp.zeros((1,), counts.dtype),
                             jnp.cumsum(counts)[:-1]])
    dest = (pstart[e_sorted] + jnp.arange(m) - start[e_sorted]).astype(jnp.int32)
    idx_pad = jnp.zeros((m_pad,), jnp.int32).at[dest].set(tok_sorted)

    x_sorted = hidden_states[idx_pad]
    y = _grouped_mlp(x_sorted, W13, W2, pstart.astype(jnp.int32),
                     counts, e, d_ff)

    inv = jnp.zeros((m,), jnp.int32).at[sort_idx].set(dest).reshape(t, K)
    out = (topk_w[:, 0:1] * y[inv[:, 0]] + topk_w[:, 1:2] * y[inv[:, 1]])
    return out.astype(hidden_states.dtype)


# speedup vs baseline: 1.3218x; 1.3218x over previous
import functools

import jax
import jax.numpy as jnp
from jax import lax
from jax.experimental import pallas as pl
from jax.experimental.pallas import tpu as pltpu

K = 2
SUB = 256
BF = 256


def _mlp_body(pstart_ref, counts_ref, x_ref, w1_ref, w3_ref, w2_ref, o_ref):
    e = pl.program_id(0)
    f = pl.program_id(1)
    p0 = pstart_ref[e]
    cnt = counts_ref[e]
    nsub = (cnt + SUB - 1) // SUB
    w1 = w1_ref[0]
    w3 = w3_ref[0]
    w2 = w2_ref[0]

    def body(j, carry):
        off = pl.multiple_of(p0 + j * SUB, SUB)
        x = x_ref[pl.ds(off, SUB), :]
        gate = lax.dot_general(x, w1, (((1,), (1,)), ((), ())),
                               preferred_element_type=jnp.float32,
                               precision=lax.Precision.DEFAULT)
        up = lax.dot_general(x, w3, (((1,), (1,)), ((), ())),
                             preferred_element_type=jnp.float32,
                             precision=lax.Precision.DEFAULT)
        h = gate * jax.nn.sigmoid(gate) * up
        part = lax.dot_general(h, w2, (((1,), (1,)), ((), ())),
                               preferred_element_type=jnp.float32,
                               precision=lax.Precision.DEFAULT)

        @pl.when(f == 0)
        def _():
            o_ref[pl.ds(off, SUB), :] = part

        @pl.when(f > 0)
        def _():
            o_ref[pl.ds(off, SUB), :] += part

        return carry

    lax.fori_loop(0, nsub, body, 0)


def _grouped_mlp(x_sorted, w13, w2, pstart, counts, n_experts, d_ff):
    m_pad, d_model = x_sorted.shape
    nf = d_ff // BF
    grid_spec = pltpu.PrefetchScalarGridSpec(
        num_scalar_prefetch=2,
        grid=(n_experts, nf),
        in_specs=[
            pl.BlockSpec((m_pad, d_model), lambda e, f, ps, ct: (0, 0)),
            pl.BlockSpec((1, BF, d_model), lambda e, f, ps, ct: (e, f, 0)),
            pl.BlockSpec((1, BF, d_model), lambda e, f, ps, ct: (e, nf + f, 0)),
            pl.BlockSpec((1, d_model, BF), lambda e, f, ps, ct: (e, 0, f)),
        ],
        out_specs=pl.BlockSpec((m_pad, d_model), lambda e, f, ps, ct: (0, 0)),
    )
    return pl.pallas_call(
        _mlp_body,
        grid_spec=grid_spec,
        out_shape=jax.ShapeDtypeStruct((m_pad, d_model), jnp.float32),
        compiler_params=pltpu.CompilerParams(
            dimension_semantics=("arbitrary", "arbitrary"),
        ),
    )(pstart, counts, x_sorted, w13, w13, w2)


def kernel(hidden_states, use_grouped_topk, top_k, router_logits,
           renormalize, W13, W2):
    t, d_model = hidden_states.shape
    e = router_logits.shape[1]
    d_ff = W2.shape[2]
    m = t * K
    m_pad = m + e * SUB

    probs = jax.nn.softmax(router_logits.astype(jnp.float32), axis=-1)
    topk_w, topk_idx = lax.top_k(probs, K)
    denom = jnp.sum(topk_w, axis=-1, keepdims=True)
    topk_w = jnp.where(jnp.asarray(renormalize), topk_w / denom, topk_w)
    topk_w = topk_w * (jnp.asarray(1, jnp.float32)
                       - jnp.asarray(use_grouped_topk, jnp.float32))

    e_flat = topk_idx.reshape(-1).astype(jnp.int32)
    sort_idx = jnp.argsort(e_flat, stable=True).astype(jnp.int32)
    tok_sorted = (sort_idx // K).astype(jnp.int32)
    e_sorted = e_flat[sort_idx]
    counts = jnp.bincount(e_flat, length=e).astype(jnp.int32)
    padded = ((counts + SUB - 1) // SUB) * SUB
    pstart = jnp.concatenate([jnp.zeros((1,), padded.dtype),
                              jnp.cumsum(padded)[:-1]])
    start = jnp.concatenate([jnp.zeros((1,), counts.dtype),
                             jnp.cumsum(counts)[:-1]])
    dest = (pstart[e_sorted] + jnp.arange(m) - start[e_sorted]).astype(jnp.int32)
    idx_pad = jnp.zeros((m_pad,), jnp.int32).at[dest].set(tok_sorted)

    x_sorted = hidden_states[idx_pad]
    y = _grouped_mlp(x_sorted, W13, W2, pstart.astype(jnp.int32),
                     counts, e, d_ff)

    inv = jnp.zeros((m,), jnp.int32).at[sort_idx].set(dest).reshape(t, K)
    out = (topk_w[:, 0:1] * y[inv[:, 0]] + topk_w[:, 1:2] * y[inv[:, 1]])
    return out.astype(hidden_states.dtype)
